# baseline (speedup 1.0000x reference)
import functools

import jax
import jax.numpy as jnp
from jax import lax
from jax.experimental import pallas as pl
from jax.experimental.pallas import tpu as pltpu
from jax.experimental.pallas import tpu_sc as plsc

NUM_CLASSES = 100000
NUM_DOMAINS = 1000
E = 128
BATCH = 16384

_info = plsc.get_sparse_core_info()
NC, NS, L = _info.num_cores, _info.num_subcores, _info.num_lanes
NW = NC * NS
ROWS_PER_W = BATCH // NW
CHUNK = 128
NCHUNK = ROWS_PER_W // CHUNK
GROUPS = CHUNK // 16


def _rsqrt16(x):
    i = lax.bitcast_convert_type(x, jnp.int32)
    i = jnp.int32(0x5F3759DF) - lax.shift_right_logical(i, 1)
    y = lax.bitcast_convert_type(i, jnp.float32)
    for _ in range(3):
        y = y * (jnp.float32(1.5) - jnp.float32(0.5) * x * y * y)
    return y


def _sc_partials(emb, gt_c, gt_d, cls_tab, dom_tab, off):
    mesh = plsc.VectorSubcoreMesh(core_axis_name="c", subcore_axis_name="s")

    @functools.partial(
        pl.kernel,
        mesh=mesh,
        out_type=jax.ShapeDtypeStruct((NW, L), jnp.float32),
        compiler_params=pltpu.CompilerParams(needs_layout_passes=False),
        scratch_types=[
            pltpu.VMEM((CHUNK,), jnp.int32),
            pltpu.VMEM((CHUNK,), jnp.int32),
            pltpu.VMEM((CHUNK, E), jnp.float32),
            pltpu.VMEM((CHUNK, E), jnp.float32),
            pltpu.VMEM((CHUNK, E), jnp.float32),
            pltpu.VMEM((E,), jnp.float32),
            pltpu.VMEM((L,), jnp.float32),
            pltpu.SemaphoreType.DMA,
            pltpu.SemaphoreType.DMA,
        ],
    )
    def k(emb_hbm, gtc_hbm, gtd_hbm, cls_hbm, dom_hbm, off_hbm, out_hbm,
          idxc_v, idxd_v, cls_v, dom_v, emb_v, off_v, acc_v, sem0, sem1):
        wid = lax.axis_index("s") * NC + lax.axis_index("c")
        pltpu.sync_copy(off_hbm, off_v)
        lane = lax.iota(jnp.int32, L)
        acc = jnp.zeros((L,), jnp.float32)
        for c in range(NCHUNK):
            base = wid * ROWS_PER_W + c * CHUNK
            pltpu.sync_copy(gtc_hbm.at[pl.ds(base, CHUNK)], idxc_v)
            pltpu.sync_copy(gtd_hbm.at[pl.ds(base, CHUNK)], idxd_v)
            cp0 = pltpu.async_copy(cls_hbm.at[idxc_v], cls_v, sem0)
            cp1 = pltpu.async_copy(dom_hbm.at[idxd_v], dom_v, sem1)
            pltpu.sync_copy(emb_hbm.at[pl.ds(base, CHUNK)], emb_v)
            cp0.wait()
            cp1.wait()

            def group(g, a):
                rows = g * 16 + lane

                def col(j, carry):
                    P, T, S = carry
                    cj = jnp.full((L,), j, jnp.int32)
                    e = plsc.load_gather(emb_v, [rows, cj])
                    cc = plsc.load_gather(cls_v, [rows, cj])
                    dd = plsc.load_gather(dom_v, [rows, cj])
                    o = plsc.load_gather(off_v, [cj])
                    pred = cc + dd + o
                    return (P + pred * pred, T + pred * e, S + e * e)

                z = jnp.zeros((L,), jnp.float32)
                P, T, S = lax.fori_loop(0, E, col, (z, z, z))
                return a + P - jnp.float32(2.0) * T * _rsqrt16(S)

            acc = lax.fori_loop(0, GROUPS, group, acc)
        acc_v[...] = acc
        pltpu.sync_copy(acc_v, out_hbm.at[wid])

    return k(emb, gt_c, gt_d, cls_tab, dom_tab, off)


def _finish(parts_ref, o_ref):
    s = jnp.sum(parts_ref[...]) * jnp.float32(1.0 / BATCH) + jnp.float32(1.0)
    o_ref[...] = jnp.full((1, 1), s, jnp.float32)


def kernel(embeddings, gt_classes, gt_domains, class_components,
           domain_components, offset_component):
    parts = _sc_partials(embeddings, gt_classes, gt_domains,
                         class_components, domain_components,
                         offset_component.reshape(E))
    out = pl.pallas_call(
        _finish,
        out_shape=jax.ShapeDtypeStruct((1, 1), jnp.float32),
    )(parts)
    return out[0, 0]


# speedup vs baseline: 1.0612x; 1.0612x over previous
import functools

import jax
import jax.numpy as jnp
from jax import lax
from jax.experimental import pallas as pl
from jax.experimental.pallas import tpu as pltpu
from jax.experimental.pallas import tpu_sc as plsc

NUM_CLASSES = 100000
NUM_DOMAINS = 1000
E = 128
BATCH = 16384

_info = plsc.get_sparse_core_info()
NC, NS, L = _info.num_cores, _info.num_subcores, _info.num_lanes
NW = NC * NS
ROWS_PER_W = BATCH // NW
CHUNK = 128
NCHUNK = ROWS_PER_W // CHUNK
GROUPS = CHUNK // 16
BLKS = E // 16


def _rsqrt16(x):
    i = lax.bitcast_convert_type(x, jnp.int32)
    i = jnp.int32(0x5F3759DF) - lax.shift_right_logical(i, 1)
    y = lax.bitcast_convert_type(i, jnp.float32)
    for _ in range(3):
        y = y * (jnp.float32(1.5) - jnp.float32(0.5) * x * y * y)
    return y


def _sc_partials(emb, gt_c, gt_d, cls_tab, dom_tab, off):
    mesh = plsc.VectorSubcoreMesh(core_axis_name="c", subcore_axis_name="s")

    @functools.partial(
        pl.kernel,
        mesh=mesh,
        out_type=jax.ShapeDtypeStruct((NW, L), jnp.float32),
        compiler_params=pltpu.CompilerParams(needs_layout_passes=False),
        scratch_types=[
            pltpu.VMEM((ROWS_PER_W,), jnp.int32),
            pltpu.VMEM((ROWS_PER_W,), jnp.int32),
            [pltpu.VMEM((CHUNK, E), jnp.float32) for _ in range(2)],
            [pltpu.VMEM((CHUNK, E), jnp.float32) for _ in range(2)],
            [pltpu.VMEM((CHUNK, E), jnp.float32) for _ in range(2)],
            pltpu.VMEM((E,), jnp.float32),
            pltpu.VMEM((L,), jnp.float32),
            [pltpu.SemaphoreType.DMA for _ in range(6)],
        ],
    )
    def k(emb_hbm, gtc_hbm, gtd_hbm, cls_hbm, dom_hbm, off_hbm, out_hbm,
          idxc_v, idxd_v, cls_b, dom_b, emb_b, off_v, acc_v, sems):
        wid = lax.axis_index("s") * NC + lax.axis_index("c")
        row0 = wid * ROWS_PER_W
        pltpu.sync_copy(gtc_hbm.at[pl.ds(row0, ROWS_PER_W)], idxc_v)
        pltpu.sync_copy(gtd_hbm.at[pl.ds(row0, ROWS_PER_W)], idxd_v)
        pltpu.sync_copy(off_hbm, off_v)

        def start(c):
            b = c % 2
            return (
                pltpu.async_copy(
                    cls_hbm.at[idxc_v.at[pl.ds(c * CHUNK, CHUNK)]],
                    cls_b[b], sems[3 * b + 0]),
                pltpu.async_copy(
                    dom_hbm.at[idxd_v.at[pl.ds(c * CHUNK, CHUNK)]],
                    dom_b[b], sems[3 * b + 1]),
                pltpu.async_copy(
                    emb_hbm.at[pl.ds(row0 + c * CHUNK, CHUNK)],
                    emb_b[b], sems[3 * b + 2]),
            )

        lane = lax.iota(jnp.int32, L)
        z = jnp.zeros((L,), jnp.float32)
        acc = z
        pending = start(0)
        for c in range(NCHUNK):
            for cp in pending:
                cp.wait()
            if c + 1 < NCHUNK:
                pending = start(c + 1)
            b = c % 2
            cls_v, dom_v, emb_v = cls_b[b], dom_b[b], emb_b[b]

            def group(g, a):
                rows = g * 16 + lane

                def blk(jj, carry):
                    P0, T0, S0, P1, T1, S1 = carry
                    jb = jj * 16
                    ov = off_v[pl.ds(jb, 16)]
                    for u in range(16):
                        cj = jnp.full((L,), jb + u, jnp.int32)
                        e = plsc.load_gather(emb_v, [rows, cj])
                        cc = plsc.load_gather(cls_v, [rows, cj])
                        dd = plsc.load_gather(dom_v, [rows, cj])
                        o = ov.at[jnp.full((L,), u, jnp.int32)].get(
                            mode="promise_in_bounds")
                        pred = cc + dd + o
                        if u % 2 == 0:
                            P0 += pred * pred
                            T0 += pred * e
                            S0 += e * e
                        else:
                            P1 += pred * pred
                            T1 += pred * e
                            S1 += e * e
                    return (P0, T0, S0, P1, T1, S1)

                P0, T0, S0, P1, T1, S1 = lax.fori_loop(
                    0, BLKS, blk, (z, z, z, z, z, z))
                P, T, S = P0 + P1, T0 + T1, S0 + S1
                return a + P - jnp.float32(2.0) * T * _rsqrt16(S)

            acc = lax.fori_loop(0, GROUPS, group, acc)
        acc_v[...] = acc
        pltpu.sync_copy(acc_v, out_hbm.at[wid])

    return k(emb, gt_c, gt_d, cls_tab, dom_tab, off)


def _finish(parts_ref, o_ref):
    s = jnp.sum(parts_ref[...]) * jnp.float32(1.0 / BATCH) + jnp.float32(1.0)
    o_ref[...] = jnp.full((1, 1), s, jnp.float32)


def kernel(embeddings, gt_classes, gt_domains, class_components,
           domain_components, offset_component):
    parts = _sc_partials(embeddings, gt_classes, gt_domains,
                         class_components, domain_components,
                         offset_component.reshape(E))
    out = pl.pallas_call(
        _finish,
        out_shape=jax.ShapeDtypeStruct((1, 1), jnp.float32),
    )(parts)
    return out[0, 0]


# speedup vs baseline: 3.6003x; 3.3926x over previous
import functools

import jax
import jax.numpy as jnp
from jax import lax
from jax.experimental import pallas as pl
from jax.experimental.pallas import tpu as pltpu
from jax.experimental.pallas import tpu_sc as plsc

NUM_CLASSES = 100000
NUM_DOMAINS = 1000
E = 128
BATCH = 16384

_info = plsc.get_sparse_core_info()
NC, NS, L = _info.num_cores, _info.num_subcores, _info.num_lanes
NW = NC * NS
ROWS_PER_W = BATCH // NW
CHUNK = 128
NCHUNK = ROWS_PER_W // CHUNK
GROUPS = CHUNK // 16
BLKS = E // 16


def _rsqrt16(x):
    i = lax.bitcast_convert_type(x, jnp.int32)
    i = jnp.int32(0x5F3759DF) - lax.shift_right_logical(i, 1)
    y = lax.bitcast_convert_type(i, jnp.float32)
    for _ in range(3):
        y = y * (jnp.float32(1.5) - jnp.float32(0.5) * x * y * y)
    return y


def _sc_partials(emb, gt_c, gt_d, cls_tab, dom_tab, off):
    mesh = plsc.VectorSubcoreMesh(core_axis_name="c", subcore_axis_name="s")

    @functools.partial(
        pl.kernel,
        mesh=mesh,
        out_type=jax.ShapeDtypeStruct((NW, L), jnp.float32),
        compiler_params=pltpu.CompilerParams(needs_layout_passes=False),
        scratch_types=[
            pltpu.VMEM((ROWS_PER_W,), jnp.int32),
            pltpu.VMEM((ROWS_PER_W,), jnp.int32),
            [pltpu.VMEM((CHUNK, E), jnp.float32) for _ in range(2)],
            [pltpu.VMEM((CHUNK, E), jnp.float32) for _ in range(2)],
            [pltpu.VMEM((CHUNK, E), jnp.float32) for _ in range(2)],
            pltpu.VMEM((E,), jnp.float32),
            pltpu.VMEM((L,), jnp.float32),
            [pltpu.SemaphoreType.DMA for _ in range(6)],
        ],
    )
    def k(emb_hbm, gtc_hbm, gtd_hbm, cls_hbm, dom_hbm, off_hbm, out_hbm,
          idxc_v, idxd_v, cls_b, dom_b, emb_b, off_v, acc_v, sems):
        wid = lax.axis_index("s") * NC + lax.axis_index("c")
        row0 = wid * ROWS_PER_W
        pltpu.sync_copy(gtc_hbm.at[pl.ds(row0, ROWS_PER_W)], idxc_v)
        pltpu.sync_copy(gtd_hbm.at[pl.ds(row0, ROWS_PER_W)], idxd_v)
        pltpu.sync_copy(off_hbm, off_v)

        def start(c):
            b = c % 2
            return (
                pltpu.async_copy(
                    cls_hbm.at[idxc_v.at[pl.ds(c * CHUNK, CHUNK)]],
                    cls_b[b], sems[3 * b + 0]),
                pltpu.async_copy(
                    dom_hbm.at[idxd_v.at[pl.ds(c * CHUNK, CHUNK)]],
                    dom_b[b], sems[3 * b + 1]),
                pltpu.async_copy(
                    emb_hbm.at[pl.ds(row0 + c * CHUNK, CHUNK)],
                    emb_b[b], sems[3 * b + 2]),
            )

        lane = lax.iota(jnp.int32, L)
        z = jnp.zeros((L,), jnp.float32)
        acc = z
        pending = start(0)
        for c in range(NCHUNK):
            for cp in pending:
                cp.wait()
            if c + 1 < NCHUNK:
                pending = start(c + 1)
            b = c % 2
            cls_v, dom_v, emb_v = cls_b[b], dom_b[b], emb_b[b]

            def group(g, a):
                rows = g * 16 + lane

                def blk(jj, carry):
                    P0, T0, S0, P1, T1, S1 = carry
                    base = jnp.full((L,), jj * 16, jnp.int32) + lane
                    for u in range(16):
                        cj = (base + u) & jnp.int32(E - 1)
                        e = plsc.load_gather(emb_v, [rows, cj])
                        cc = plsc.load_gather(cls_v, [rows, cj])
                        dd = plsc.load_gather(dom_v, [rows, cj])
                        o = plsc.load_gather(off_v, [cj])
                        pred = cc + dd + o
                        if u % 2 == 0:
                            P0 += pred * pred
                            T0 += pred * e
                            S0 += e * e
                        else:
                            P1 += pred * pred
                            T1 += pred * e
                            S1 += e * e
                    return (P0, T0, S0, P1, T1, S1)

                P0, T0, S0, P1, T1, S1 = lax.fori_loop(
                    0, BLKS, blk, (z, z, z, z, z, z))
                P, T, S = P0 + P1, T0 + T1, S0 + S1
                return a + P - jnp.float32(2.0) * T * _rsqrt16(S)

            acc = lax.fori_loop(0, GROUPS, group, acc)
        acc_v[...] = acc
        pltpu.sync_copy(acc_v, out_hbm.at[wid])

    return k(emb, gt_c, gt_d, cls_tab, dom_tab, off)


def _finish(parts_ref, o_ref):
    s = jnp.sum(parts_ref[...]) * jnp.float32(1.0 / BATCH) + jnp.float32(1.0)
    o_ref[...] = jnp.full((1, 1), s, jnp.float32)


def kernel(embeddings, gt_classes, gt_domains, class_components,
           domain_components, offset_component):
    parts = _sc_partials(embeddings, gt_classes, gt_domains,
                         class_components, domain_components,
                         offset_component.reshape(E))
    out = pl.pallas_call(
        _finish,
        out_shape=jax.ShapeDtypeStruct((1, 1), jnp.float32),
    )(parts)
    return out[0, 0]
